# baseline (speedup 1.0000x reference)
import functools

import jax
import jax.numpy as jnp
from jax import lax
from jax.experimental import pallas as pl
from jax.experimental.pallas import tpu as pltpu
from jax.experimental.pallas import tpu_sc as plsc

_NC = 2
_NS = 16
_NW = _NC * _NS
_L = 16

_B = 1000
_BP = 1024
_BW = _BP // _NW
_NNB = 8
_NBW = _BW * _NNB
_NBH = _NBW // 2
_KH = _NNB // 2
_DIM = 512
_EPS = 0.01
_NRELP = 64
_NCTX = 16
_EPR = 8


def _sc_gather(u_pad, v_pad, adj_cat, usr, ent):
    mesh = plsc.VectorSubcoreMesh(core_axis_name="c", subcore_axis_name="s")
    _CH = 64
    _NCHN = _NBW // _CH

    @functools.partial(
        pl.kernel,
        mesh=mesh,
        compiler_params=pltpu.CompilerParams(needs_layout_passes=False),
        out_type=(
            jax.ShapeDtypeStruct((_BP, _DIM), jnp.float32),
            jax.ShapeDtypeStruct((_BP, _DIM), jnp.float32),
            jax.ShapeDtypeStruct((_BP * _NNB,), jnp.int32),
            jax.ShapeDtypeStruct((_BP * _NNB, _DIM), jnp.float32),
        ),
        scratch_types=[
            pltpu.VMEM((_BW,), jnp.int32),
            pltpu.VMEM((_BW,), jnp.int32),
            pltpu.VMEM((_BW,), jnp.int32),
            pltpu.VMEM((_BW,), jnp.int32),
            pltpu.VMEM((_BW, 128), jnp.int32),
            pltpu.VMEM((_CH,), jnp.int32),
            pltpu.VMEM((_CH,), jnp.int32),
            pltpu.VMEM((_CH,), jnp.int32),
            pltpu.VMEM((_CH,), jnp.int32),
            pltpu.VMEM((_NBW,), jnp.int32),
            pltpu.VMEM((_CH, _DIM), jnp.float32),
            pltpu.VMEM((_CH, _DIM), jnp.float32),
            pltpu.VMEM((_CH, _DIM), jnp.float32),
            pltpu.SemaphoreType.DMA,
            pltpu.SemaphoreType.DMA,
            pltpu.SemaphoreType.DMA,
            pltpu.SemaphoreType.DMA,
            pltpu.SemaphoreType.DMA,
            pltpu.SemaphoreType.DMA,
            pltpu.SemaphoreType.DMA,
            pltpu.SemaphoreType.DMA,
        ],
    )
    def k(u_hbm, v_hbm, adj_hbm, usr_hbm, ent_hbm,
          uemb_out, self_out, nbrel_out, nbvec_out,
          uidx, vidx, vgidx, vmidx, adj_rows, fl0, fl1, fl2, fl3, frel,
          buf_a, buf_b, buf_c,
          sem_a, sem_gu, sem_g0, sem_g1, sem_ou, sem_o0, sem_o1, sem_f):
        wid = lax.axis_index("s") * _NC + lax.axis_index("c")
        base = wid * _BW
        pltpu.sync_copy(u_hbm.at[pl.ds(base, _BW)], uidx)
        pltpu.sync_copy(v_hbm.at[pl.ds(base, _BW)], vidx)
        for h in range(_BW // _L):
            vv = vidx[pl.ds(h * _L, _L)]
            vgidx[pl.ds(h * _L, _L)] = lax.shift_right_logical(vv, 3)
            vmidx[pl.ds(h * _L, _L)] = lax.shift_left(
                lax.bitwise_and(vv, 7), 4)
        ca = pltpu.async_copy(adj_hbm.at[vgidx], adj_rows, sem_a)
        gu = pltpu.async_copy(usr_hbm.at[uidx], buf_a.at[pl.ds(0, _BW)],
                              sem_gu)
        gv = pltpu.async_copy(ent_hbm.at[vidx], buf_a.at[pl.ds(_BW, _BW)],
                              sem_gu)
        ca.wait()
        lane = lax.iota(jnp.int32, _L)
        rowoff = lax.shift_right_logical(lane, 3)
        koff = lax.bitwise_and(lane, 7)
        fls = [fl0, fl1, fl2, fl3]
        gn = [None] * _NCHN
        bufs = [buf_b, buf_c, buf_a, buf_b]
        gsems = [sem_g0, sem_g1, sem_gu, sem_g0]
        for c in range(_NCHN):
            for j in range(_CH // _L):
                rows_j = rowoff + (c * (_CH // _L) + j) * 2
                off = plsc.load_gather(vmidx, [rows_j]) + koff
                fls[c][pl.ds(j * _L, _L)] = plsc.load_gather(
                    adj_rows, [rows_j, off])
            if c < 2:
                gn[c] = pltpu.async_copy(ent_hbm.at[fls[c]], bufs[c],
                                         gsems[c])
        for jj in range(_NBW // _L):
            rows_jj = rowoff + jj * 2
            off = plsc.load_gather(vmidx, [rows_jj]) + koff + _NNB
            frel[pl.ds(jj * _L, _L)] = plsc.load_gather(adj_rows,
                                                        [rows_jj, off])
        cf = pltpu.async_copy(frel, nbrel_out.at[pl.ds(wid * _NBW, _NBW)],
                              sem_f)
        gu.wait()
        gv.wait()
        ou = pltpu.async_copy(buf_a.at[pl.ds(0, _BW)],
                              uemb_out.at[pl.ds(base, _BW)], sem_ou)
        ov = pltpu.async_copy(buf_a.at[pl.ds(_BW, _BW)],
                              self_out.at[pl.ds(base, _BW)], sem_ou)
        osems = [sem_o0, sem_o1, sem_ou, sem_o0]
        on = [None] * _NCHN
        nbase = base * _NNB
        for c in range(_NCHN):
            if c == 2:
                ou.wait()
                ov.wait()
                gn[2] = pltpu.async_copy(ent_hbm.at[fls[2]], bufs[2],
                                         gsems[2])
            if c == 3:
                on[0].wait()
                gn[3] = pltpu.async_copy(ent_hbm.at[fls[3]], bufs[3],
                                         gsems[3])
            gn[c].wait()
            on[c] = pltpu.async_copy(
                bufs[c], nbvec_out.at[pl.ds(nbase + c * _CH, _CH)], osems[c])
        on[1].wait()
        on[2].wait()
        on[3].wait()
        cf.wait()

    return k(u_pad, v_pad, adj_cat, usr, ent)


def _tc_compute(user_emb, self_vec, nb_vec, nb_rel, relT, W_aggT, W_linP):
    BM = 256
    grid = (_BP // BM,)

    def body(user_ref, self_ref, nb_ref, nbr_ref, relT_ref,
             wagg_ref, wlin_ref, fea_ref, feaa_ref):
        i = pl.program_id(0)
        user = user_ref[...]
        s_all = jnp.dot(user, relT_ref[...], preferred_element_type=jnp.float32)
        nbr = nbr_ref[...]
        r_iota = lax.broadcasted_iota(jnp.int32, (BM, _NRELP), 1)
        cols = []
        for kk in range(_NNB):
            sel = nbr[:, kk:kk + 1] == r_iota
            cols.append(jnp.sum(jnp.where(sel, s_all, 0.0), axis=1,
                                keepdims=True))
        scores = jnp.concatenate(cols, axis=1)
        m = jnp.max(scores, axis=-1, keepdims=True)
        e = jnp.exp(scores - m)
        w = e / jnp.sum(e, axis=-1, keepdims=True)
        agg = w[:, 0:1] * nb_ref[:, 0:_DIM]
        for kk in range(1, _NNB):
            agg = agg + w[:, kk:kk + 1] * nb_ref[:, kk * _DIM:(kk + 1) * _DIM]
        x = self_ref[...] + agg
        item = jnp.tanh(jnp.dot(x, wagg_ref[...],
                                preferred_element_type=jnp.float32))
        xp = x + jnp.sign(agg) * _EPS
        item2 = jnp.tanh(jnp.dot(xp, wagg_ref[...],
                                 preferred_element_type=jnp.float32))
        wl = wlin_ref[...]
        fa = jnp.dot(wl, item, preferred_element_type=jnp.float32)
        fb = jnp.dot(wl, item2, preferred_element_type=jnp.float32)

        @pl.when(i == 0)
        def _():
            fea_ref[...] = jnp.zeros_like(fea_ref)
            feaa_ref[...] = jnp.zeros_like(feaa_ref)

        fea_ref[...] += fa
        feaa_ref[...] += fb

    return pl.pallas_call(
        body,
        grid=grid,
        in_specs=[
            pl.BlockSpec((BM, _DIM), lambda i: (i, 0)),
            pl.BlockSpec((BM, _DIM), lambda i: (i, 0)),
            pl.BlockSpec((BM, _NNB * _DIM), lambda i: (i, 0)),
            pl.BlockSpec((BM, _NNB), lambda i: (i, 0)),
            pl.BlockSpec((_DIM, _NRELP), lambda i: (0, 0)),
            pl.BlockSpec((_DIM, _DIM), lambda i: (0, 0)),
            pl.BlockSpec((_NCTX, BM), lambda i: (0, i)),
        ],
        out_specs=[
            pl.BlockSpec((_NCTX, _DIM), lambda i: (0, 0)),
            pl.BlockSpec((_NCTX, _DIM), lambda i: (0, 0)),
        ],
        out_shape=[
            jax.ShapeDtypeStruct((_NCTX, _DIM), jnp.float32),
            jax.ShapeDtypeStruct((_NCTX, _DIM), jnp.float32),
        ],
    )(user_emb, self_vec, nb_vec, nb_rel, relT, W_aggT, W_linP)


def kernel(u, v, adj_ent, adj_rel, usr, ent, rel, W_agg, W_lin):
    bsz = u.shape[0]
    u_pad = jnp.zeros((_BP,), jnp.int32).at[:bsz].set(u.astype(jnp.int32))
    v_pad = jnp.zeros((_BP,), jnp.int32).at[:bsz].set(v.astype(jnp.int32))
    adj_cat = jnp.concatenate(
        [adj_ent.astype(jnp.int32), adj_rel.astype(jnp.int32)],
        axis=1).reshape(-1, 128)

    uemb, selfv, nbrel_flat, nbvec_flat = _sc_gather(
        u_pad, v_pad, adj_cat, usr, ent)

    nbvec = nbvec_flat.reshape(_BP, _NNB * _DIM)
    nb_rel = nbrel_flat.reshape(_BP, _NNB)
    relT = jnp.zeros((_DIM, _NRELP), jnp.float32).at[:, :rel.shape[0]].set(rel.T)
    W_linP = jnp.zeros((_NCTX, _BP), jnp.float32).at[:, :bsz].set(W_lin)

    fea, fea_agg = _tc_compute(uemb, selfv, nbvec, nb_rel,
                               relT, W_agg.T, W_linP)
    return fea, fea_agg
